# baseline (speedup 1.0000x reference)
import functools

import jax
import jax.numpy as jnp
from jax import lax
from jax.experimental import pallas as pl
from jax.experimental.pallas import tpu as pltpu
from jax.experimental.pallas import tpu_sc as plsc

VOCAB = 100000
HID = 128
MAXPOS = 512
VDIM = 2048
B = 1024
S = 200
NB = 36
EPS = 1e-12

L = 16
NC = 2
NS = 16
NW = NC * NS
TOTAL = B * S
PER_W = TOTAL // NW
CHUNK = 128
NCHUNK = PER_W // CHUNK
NPAIR = NCHUNK // 2
NV = HID // L


def _text_sum_sc(tid, pid, tt, word_emb, pos_emb):
    mesh = plsc.VectorSubcoreMesh(core_axis_name="c", subcore_axis_name="s")

    @functools.partial(
        pl.kernel,
        out_type=jax.ShapeDtypeStruct((TOTAL, HID), jnp.float32),
        mesh=mesh,
        scratch_types=[
            pltpu.VMEM((2, CHUNK), jnp.int32),
            pltpu.VMEM((2, CHUNK), jnp.int32),
            pltpu.VMEM((2, CHUNK), jnp.int32),
            pltpu.VMEM((2, CHUNK, HID), jnp.float32),
            pltpu.VMEM((2, CHUNK, HID), jnp.float32),
            pltpu.VMEM((2, CHUNK, HID), jnp.float32),
            pltpu.SemaphoreType.DMA,
            pltpu.SemaphoreType.DMA,
            pltpu.SemaphoreType.DMA,
            pltpu.SemaphoreType.DMA,
        ],
    )
    def text_kernel(tid_h, pid_h, tt_h, wtab_h, ptab_h, out_h,
                    tid_v, pid_v, tt_v, wbuf, pbuf, tbuf,
                    gs0, gs1, os0, os1):
        wid = lax.axis_index("s") * NC + lax.axis_index("c")
        base_w = wid * PER_W
        gsems = (gs0, gs1)
        osems = (os0, os1)

        def gather_descs(c, b):
            base = base_w + c * CHUNK
            del base
            return (
                pltpu.make_async_copy(wtab_h.at[tid_v.at[b]], wbuf.at[b], gsems[b]),
                pltpu.make_async_copy(ptab_h.at[pid_v.at[b]], pbuf.at[b], gsems[b]),
                pltpu.make_async_copy(wtab_h.at[tt_v.at[b]], tbuf.at[b], gsems[b]),
            )

        def fire(c, b):
            base = base_w + c * CHUNK
            pltpu.sync_copy(tid_h.at[pl.ds(base, CHUNK)], tid_v.at[b])
            pltpu.sync_copy(pid_h.at[pl.ds(base, CHUNK)], pid_v.at[b])
            pltpu.sync_copy(tt_h.at[pl.ds(base, CHUNK)], tt_v.at[b])
            for d in gather_descs(c, b):
                d.start()

        def wait_gathers(c, b):
            for d in gather_descs(c, b):
                d.wait()

        def compute(b):
            def row_body(i, carry):
                for j in range(NV):
                    sl = pl.ds(j * L, L)
                    wbuf[b, i, sl] = (wbuf[b, i, sl] + pbuf[b, i, sl]
                                      + tbuf[b, i, sl])
                return carry

            lax.fori_loop(0, CHUNK, row_body, 0, unroll=4)

        def start_out(c, b):
            base = base_w + c * CHUNK
            pltpu.make_async_copy(wbuf.at[b], out_h.at[pl.ds(base, CHUNK)],
                                  osems[b]).start()

        def wait_out(c, b):
            base = base_w + c * CHUNK
            pltpu.make_async_copy(wbuf.at[b], out_h.at[pl.ds(base, CHUNK)],
                                  osems[b]).wait()

        fire(0, 0)

        def pair_body(p, carry):
            c0 = 2 * p
            c1 = c0 + 1

            @pl.when(p > 0)
            def _():
                wait_out(c0 - 1, 1)

            fire(c1, 1)
            wait_gathers(c0, 0)
            compute(0)
            start_out(c0, 0)

            @pl.when(p + 1 < NPAIR)
            def _():
                wait_out(c0, 0)
                fire(c0 + 2, 0)

            wait_gathers(c1, 1)
            compute(1)
            start_out(c1, 1)
            return carry

        lax.fori_loop(0, NPAIR, pair_body, 0, unroll=False)
        wait_out(NCHUNK - 2, 0)
        wait_out(NCHUNK - 1, 1)

    return text_kernel(tid, pid, tt, word_emb, pos_emb)


def _ln_tc(y, g, b):
    mu = jnp.mean(y, axis=-1, keepdims=True)
    d = y - mu
    var = jnp.mean(d * d, axis=-1, keepdims=True)
    return d * lax.rsqrt(var + EPS) * g + b


def _text_ln_tc(x, g, b):
    BM = 1024

    def body(x_ref, g_ref, b_ref, o_ref):
        o_ref[...] = _ln_tc(x_ref[...], g_ref[...], b_ref[...])

    row_spec = pl.BlockSpec((1, HID), lambda i: (0, 0))
    return pl.pallas_call(
        body,
        grid=(TOTAL // BM,),
        in_specs=[pl.BlockSpec((BM, HID), lambda i: (i, 0)), row_spec, row_spec],
        out_specs=pl.BlockSpec((BM, HID), lambda i: (i, 0)),
        out_shape=jax.ShapeDtypeStruct((TOTAL, HID), jnp.float32),
    )(x, g, b)


def _image_tc(image_flat, img_W, img_b, iln_g, iln_b, w1row, vln_g, vln_b):
    M = B * NB
    BM = 512

    def body(x_ref, w_ref, b_ref, ig_ref, ib_ref, w1_ref, vg_ref, vb_ref, o_ref):
        y = jnp.dot(x_ref[...], w_ref[...], preferred_element_type=jnp.float32)
        y = y + b_ref[...]
        y = _ln_tc(y, ig_ref[...], ib_ref[...])
        y = y + w1_ref[...]
        o_ref[...] = _ln_tc(y, vg_ref[...], vb_ref[...])

    row_spec = pl.BlockSpec((1, HID), lambda i: (0, 0))
    return pl.pallas_call(
        body,
        grid=(M // BM,),
        in_specs=[
            pl.BlockSpec((BM, VDIM), lambda i: (i, 0)),
            pl.BlockSpec((VDIM, HID), lambda i: (0, 0)),
            row_spec, row_spec, row_spec, row_spec, row_spec, row_spec,
        ],
        out_specs=pl.BlockSpec((BM, HID), lambda i: (i, 0)),
        out_shape=jax.ShapeDtypeStruct((M, HID), jnp.float32),
    )(image_flat, img_W, img_b, iln_g, iln_b, w1row, vln_g, vln_b)


def kernel(token_ids, image_feat, token_type_ids, position_ids, word_emb,
           pos_emb, img_W, img_b, ln_g, ln_b, iln_g, iln_b, vln_g, vln_b):
    tid = token_ids.reshape(-1)
    pid = position_ids.reshape(-1)
    tt = token_type_ids.reshape(-1)
    ssum = _text_sum_sc(tid, pid, tt, word_emb, pos_emb)
    r = lambda a: a.reshape(1, HID)
    emb = _text_ln_tc(ssum, r(ln_g), r(ln_b)).reshape(B, S, HID)

    w1row = lax.slice(word_emb, (1, 0), (2, HID))
    v = _image_tc(image_feat.reshape(B * NB, VDIM), img_W, r(img_b),
                  r(iln_g), r(iln_b), w1row, r(vln_g), r(vln_b))
    return (emb, v.reshape(B, NB, HID))


# speedup vs baseline: 4.7617x; 4.7617x over previous
import functools

import jax
import jax.numpy as jnp
from jax import lax
from jax.experimental import pallas as pl
from jax.experimental.pallas import tpu as pltpu
from jax.experimental.pallas import tpu_sc as plsc

VOCAB = 100000
HID = 128
MAXPOS = 512
VDIM = 2048
B = 1024
S = 200
NB = 36
EPS = 1e-12

NC = 2
NS = 16
NW = NC * NS
TOTAL = B * S
PER_W = TOTAL // NW
CHUNK = 128
NCHUNK = PER_W // CHUNK
NPAIR = NCHUNK // 2


def _word_gather_sc(tid, word_emb):
    mesh = plsc.VectorSubcoreMesh(core_axis_name="c", subcore_axis_name="s")

    @functools.partial(
        pl.kernel,
        out_type=jax.ShapeDtypeStruct((TOTAL, HID), jnp.float32),
        mesh=mesh,
        scratch_types=[
            pltpu.VMEM((2, CHUNK), jnp.int32),
            pltpu.VMEM((2, CHUNK, HID), jnp.float32),
            pltpu.SemaphoreType.DMA,
            pltpu.SemaphoreType.DMA,
            pltpu.SemaphoreType.DMA,
            pltpu.SemaphoreType.DMA,
        ],
    )
    def gather_kernel(tid_h, wtab_h, out_h, tid_v, wbuf, gs0, gs1, os0, os1):
        wid = lax.axis_index("s") * NC + lax.axis_index("c")
        base_w = wid * PER_W
        gsems = (gs0, gs1)
        osems = (os0, os1)

        def gather_desc(b):
            return pltpu.make_async_copy(wtab_h.at[tid_v.at[b]], wbuf.at[b],
                                         gsems[b])

        def fire(c, b):
            base = base_w + c * CHUNK
            pltpu.sync_copy(tid_h.at[pl.ds(base, CHUNK)], tid_v.at[b])
            gather_desc(b).start()

        def out_desc(c, b):
            base = base_w + c * CHUNK
            return pltpu.make_async_copy(wbuf.at[b], out_h.at[pl.ds(base, CHUNK)],
                                         osems[b])

        fire(0, 0)

        def pair_body(p, carry):
            c0 = 2 * p
            c1 = c0 + 1

            @pl.when(p > 0)
            def _():
                out_desc(c0 - 1, 1).wait()

            fire(c1, 1)
            gather_desc(0).wait()
            out_desc(c0, 0).start()

            @pl.when(p + 1 < NPAIR)
            def _():
                out_desc(c0, 0).wait()
                fire(c0 + 2, 0)

            gather_desc(1).wait()
            out_desc(c1, 1).start()
            return carry

        lax.fori_loop(0, NPAIR, pair_body, 0, unroll=False)
        out_desc(NCHUNK - 2, 0).wait()
        out_desc(NCHUNK - 1, 1).wait()

    return gather_kernel(tid, word_emb)


def _ln_tc(y, g, b):
    mu = jnp.mean(y, axis=-1, keepdims=True)
    d = y - mu
    var = jnp.mean(d * d, axis=-1, keepdims=True)
    return d * lax.rsqrt(var + EPS) * g + b


def _text_tc(wrows, pid, tt, pos_emb, w01, g, b):
    BM = 1024

    def body(w_ref, pid_ref, tt_ref, ptab_ref, w01_ref, g_ref, b_ref, o_ref):
        pid_col = pid_ref[...]
        onehot = (pid_col == lax.broadcasted_iota(jnp.int32, (BM, MAXPOS), 1)
                  ).astype(jnp.float32)
        pos = jnp.dot(onehot, ptab_ref[...], preferred_element_type=jnp.float32)
        t = tt_ref[...].astype(jnp.float32)
        row0 = w01_ref[0:1, :]
        row1 = w01_ref[1:2, :]
        ttemb = row0 + t * (row1 - row0)
        o_ref[...] = _ln_tc(w_ref[...] + pos + ttemb, g_ref[...], b_ref[...])

    row_spec = pl.BlockSpec((1, HID), lambda i: (0, 0))
    return pl.pallas_call(
        body,
        grid=(TOTAL // BM,),
        in_specs=[
            pl.BlockSpec((BM, HID), lambda i: (i, 0)),
            pl.BlockSpec((BM, 1), lambda i: (i, 0)),
            pl.BlockSpec((BM, 1), lambda i: (i, 0)),
            pl.BlockSpec((MAXPOS, HID), lambda i: (0, 0)),
            pl.BlockSpec((2, HID), lambda i: (0, 0)),
            row_spec, row_spec,
        ],
        out_specs=pl.BlockSpec((BM, HID), lambda i: (i, 0)),
        out_shape=jax.ShapeDtypeStruct((TOTAL, HID), jnp.float32),
    )(wrows, pid, tt, pos_emb, w01, g, b)


def _image_tc(image_flat, img_W, img_b, iln_g, iln_b, w1row, vln_g, vln_b):
    M = B * NB
    BM = 512

    def body(x_ref, w_ref, b_ref, ig_ref, ib_ref, w1_ref, vg_ref, vb_ref, o_ref):
        y = jnp.dot(x_ref[...], w_ref[...], preferred_element_type=jnp.float32)
        y = y + b_ref[...]
        y = _ln_tc(y, ig_ref[...], ib_ref[...])
        y = y + w1_ref[...]
        o_ref[...] = _ln_tc(y, vg_ref[...], vb_ref[...])

    row_spec = pl.BlockSpec((1, HID), lambda i: (0, 0))
    return pl.pallas_call(
        body,
        grid=(M // BM,),
        in_specs=[
            pl.BlockSpec((BM, VDIM), lambda i: (i, 0)),
            pl.BlockSpec((VDIM, HID), lambda i: (0, 0)),
            row_spec, row_spec, row_spec, row_spec, row_spec, row_spec,
        ],
        out_specs=pl.BlockSpec((BM, HID), lambda i: (i, 0)),
        out_shape=jax.ShapeDtypeStruct((M, HID), jnp.float32),
    )(image_flat, img_W, img_b, iln_g, iln_b, w1row, vln_g, vln_b)


def kernel(token_ids, image_feat, token_type_ids, position_ids, word_emb,
           pos_emb, img_W, img_b, ln_g, ln_b, iln_g, iln_b, vln_g, vln_b):
    tid = token_ids.reshape(-1)
    wrows = _word_gather_sc(tid, word_emb)

    r = lambda a: a.reshape(1, HID)
    w01 = lax.slice(word_emb, (0, 0), (2, HID))
    w1row = lax.slice(word_emb, (1, 0), (2, HID))
    v = _image_tc(image_feat.reshape(B * NB, VDIM), img_W, r(img_b),
                  r(iln_g), r(iln_b), w1row, r(vln_g), r(vln_b))

    emb = _text_tc(wrows, position_ids.reshape(TOTAL, 1),
                   token_type_ids.reshape(TOTAL, 1), pos_emb, w01,
                   r(ln_g), r(ln_b)).reshape(B, S, HID)
    return (emb, v.reshape(B, NB, HID))


# speedup vs baseline: 9.1051x; 1.9122x over previous
import functools

import jax
import jax.numpy as jnp
from jax import lax
from jax.experimental import pallas as pl
from jax.experimental.pallas import tpu as pltpu
from jax.experimental.pallas import tpu_sc as plsc

VOCAB = 100000
HID = 128
MAXPOS = 512
VDIM = 2048
B = 1024
S = 200
NB = 36
EPS = 1e-12

NC = 2
NS = 16
NW = NC * NS
TOTAL = B * S
PER_W = TOTAL // NW
CHUNK = 128
NCHUNK = PER_W // CHUNK
NPAIR = NCHUNK // 2


def _word_gather_sc(tid, word_emb):
    mesh = plsc.VectorSubcoreMesh(core_axis_name="c", subcore_axis_name="s")

    @functools.partial(
        pl.kernel,
        out_type=jax.ShapeDtypeStruct((TOTAL, HID), jnp.float32),
        mesh=mesh,
        scratch_types=[
            pltpu.VMEM((2, CHUNK), jnp.int32),
            pltpu.VMEM((2, CHUNK, HID), jnp.float32),
            pltpu.SemaphoreType.DMA,
            pltpu.SemaphoreType.DMA,
            pltpu.SemaphoreType.DMA,
            pltpu.SemaphoreType.DMA,
        ],
    )
    def gather_kernel(tid_h, wtab_h, out_h, tid_v, wbuf, gs0, gs1, os0, os1):
        wid = lax.axis_index("s") * NC + lax.axis_index("c")
        base_w = wid * PER_W
        gsems = (gs0, gs1)
        osems = (os0, os1)

        def gather_desc(b):
            return pltpu.make_async_copy(wtab_h.at[tid_v.at[b]], wbuf.at[b],
                                         gsems[b])

        def fire(c, b):
            base = base_w + c * CHUNK
            pltpu.sync_copy(tid_h.at[pl.ds(base, CHUNK)], tid_v.at[b])
            gather_desc(b).start()

        def out_desc(c, b):
            base = base_w + c * CHUNK
            return pltpu.make_async_copy(wbuf.at[b], out_h.at[pl.ds(base, CHUNK)],
                                         osems[b])

        fire(0, 0)

        def pair_body(p, carry):
            c0 = 2 * p
            c1 = c0 + 1

            @pl.when(p > 0)
            def _():
                out_desc(c0 - 1, 1).wait()

            fire(c1, 1)
            gather_desc(0).wait()
            out_desc(c0, 0).start()

            @pl.when(p + 1 < NPAIR)
            def _():
                out_desc(c0, 0).wait()
                fire(c0 + 2, 0)

            gather_desc(1).wait()
            out_desc(c1, 1).start()
            return carry

        lax.fori_loop(0, NPAIR, pair_body, 0, unroll=False)
        out_desc(NCHUNK - 2, 0).wait()
        out_desc(NCHUNK - 1, 1).wait()

    return gather_kernel(tid, word_emb)


def _ln_tc(y, g, b):
    mu = jnp.mean(y, axis=-1, keepdims=True)
    d = y - mu
    var = jnp.mean(d * d, axis=-1, keepdims=True)
    return d * lax.rsqrt(var + EPS) * g + b


def _text_tc(wrows, pid, tt, pos_emb, w01, g, b):
    BM = 1024

    def body(w_ref, pid_ref, tt_ref, ptab_ref, w01_ref, g_ref, b_ref, o_ref):
        pid_col = pid_ref[...]
        onehot = (pid_col == lax.broadcasted_iota(jnp.int32, (BM, MAXPOS), 1)
                  ).astype(jnp.float32)
        pos = jnp.dot(onehot, ptab_ref[...], preferred_element_type=jnp.float32)
        t = tt_ref[...].astype(jnp.float32)
        row0 = w01_ref[0:1, :]
        row1 = w01_ref[1:2, :]
        ttemb = row0 + t * (row1 - row0)
        o_ref[...] = _ln_tc(w_ref[...] + pos + ttemb, g_ref[...], b_ref[...])

    row_spec = pl.BlockSpec((1, HID), lambda i: (0, 0))
    return pl.pallas_call(
        body,
        grid=(TOTAL // BM,),
        in_specs=[
            pl.BlockSpec((BM, HID), lambda i: (i, 0)),
            pl.BlockSpec((BM, 1), lambda i: (i, 0)),
            pl.BlockSpec((BM, 1), lambda i: (i, 0)),
            pl.BlockSpec((MAXPOS, HID), lambda i: (0, 0)),
            pl.BlockSpec((2, HID), lambda i: (0, 0)),
            row_spec, row_spec,
        ],
        out_specs=pl.BlockSpec((BM, HID), lambda i: (i, 0)),
        out_shape=jax.ShapeDtypeStruct((TOTAL, HID), jnp.float32),
    )(wrows, pid, tt, pos_emb, w01, g, b)


def _image_tc(image_flat, img_W, img_b, iln_g, iln_b, w1row, vln_g, vln_b):
    M = B * NB
    BM = 512

    def body(x_ref, w_ref, b_ref, ig_ref, ib_ref, w1_ref, vg_ref, vb_ref, o_ref):
        y = jnp.dot(x_ref[...], w_ref[...], preferred_element_type=jnp.float32)
        y = y + b_ref[...]
        y = _ln_tc(y, ig_ref[...], ib_ref[...])
        y = y + w1_ref[...]
        o_ref[...] = _ln_tc(y, vg_ref[...], vb_ref[...])

    row_spec = pl.BlockSpec((1, HID), lambda i: (0, 0))
    return pl.pallas_call(
        body,
        grid=(M // BM,),
        in_specs=[
            pl.BlockSpec((BM, VDIM), lambda i: (i, 0)),
            pl.BlockSpec((VDIM, HID), lambda i: (0, 0)),
            row_spec, row_spec, row_spec, row_spec, row_spec, row_spec,
        ],
        out_specs=pl.BlockSpec((BM, HID), lambda i: (i, 0)),
        out_shape=jax.ShapeDtypeStruct((M, HID), jnp.float32),
    )(image_flat, img_W, img_b, iln_g, iln_b, w1row, vln_g, vln_b)


def kernel(token_ids, image_feat, token_type_ids, position_ids, word_emb,
           pos_emb, img_W, img_b, ln_g, ln_b, iln_g, iln_b, vln_g, vln_b):
    tid = token_ids.reshape(-1)
    wrows = _word_gather_sc(tid, word_emb)

    r = lambda a: a.reshape(1, HID)
    w01 = lax.slice(word_emb, (0, 0), (2, HID))
    w1row = lax.slice(word_emb, (1, 0), (2, HID))
    img_rows = image_feat.transpose(1, 0, 2).reshape(NB * B, VDIM)
    v = _image_tc(img_rows, img_W, r(img_b),
                  r(iln_g), r(iln_b), w1row, r(vln_g), r(vln_b))
    v = v.reshape(NB, B, HID).transpose(1, 0, 2)

    emb = _text_tc(wrows, position_ids.reshape(TOTAL, 1),
                   token_type_ids.reshape(TOTAL, 1), pos_emb, w01,
                   r(ln_g), r(ln_b)).reshape(B, S, HID)
    return (emb, v)


# speedup vs baseline: 14.0507x; 1.5432x over previous
import functools

import jax
import jax.numpy as jnp
from jax import lax
from jax.experimental import pallas as pl
from jax.experimental.pallas import tpu as pltpu
from jax.experimental.pallas import tpu_sc as plsc

VOCAB = 100000
HID = 128
MAXPOS = 512
VDIM = 2048
B = 1024
S = 200
NB = 36
EPS = 1e-12

NC = 2
NS = 16
NW = NC * NS
TOTAL = B * S
PER_W = TOTAL // NW
CHUNK = 128
NCHUNK = PER_W // CHUNK
NPAIR = NCHUNK // 2


def _word_gather_sc(tid, word_emb):
    mesh = plsc.VectorSubcoreMesh(core_axis_name="c", subcore_axis_name="s")

    @functools.partial(
        pl.kernel,
        out_type=jax.ShapeDtypeStruct((TOTAL, HID), jnp.float32),
        mesh=mesh,
        scratch_types=[
            pltpu.VMEM((2, CHUNK), jnp.int32),
            pltpu.VMEM((2, CHUNK, HID), jnp.float32),
            pltpu.SemaphoreType.DMA,
            pltpu.SemaphoreType.DMA,
            pltpu.SemaphoreType.DMA,
            pltpu.SemaphoreType.DMA,
        ],
    )
    def gather_kernel(tid_h, wtab_h, out_h, tid_v, wbuf, gs0, gs1, os0, os1):
        wid = lax.axis_index("s") * NC + lax.axis_index("c")
        base_w = wid * PER_W
        gsems = (gs0, gs1)
        osems = (os0, os1)

        def gather_desc(b):
            return pltpu.make_async_copy(wtab_h.at[tid_v.at[b]], wbuf.at[b],
                                         gsems[b])

        def fire(c, b):
            base = base_w + c * CHUNK
            pltpu.sync_copy(tid_h.at[pl.ds(base, CHUNK)], tid_v.at[b])
            gather_desc(b).start()

        def out_desc(c, b):
            base = base_w + c * CHUNK
            return pltpu.make_async_copy(wbuf.at[b], out_h.at[pl.ds(base, CHUNK)],
                                         osems[b])

        fire(0, 0)

        def pair_body(p, carry):
            c0 = 2 * p
            c1 = c0 + 1

            @pl.when(p > 0)
            def _():
                out_desc(c0 - 1, 1).wait()

            fire(c1, 1)
            gather_desc(0).wait()
            out_desc(c0, 0).start()

            @pl.when(p + 1 < NPAIR)
            def _():
                out_desc(c0, 0).wait()
                fire(c0 + 2, 0)

            gather_desc(1).wait()
            out_desc(c1, 1).start()
            return carry

        lax.fori_loop(0, NPAIR, pair_body, 0, unroll=False)
        out_desc(NCHUNK - 2, 0).wait()
        out_desc(NCHUNK - 1, 1).wait()

    return gather_kernel(tid, word_emb)


def _ln_tc(y, g, b):
    mu = jnp.mean(y, axis=-1, keepdims=True)
    d = y - mu
    var = jnp.mean(d * d, axis=-1, keepdims=True)
    return d * lax.rsqrt(var + EPS) * g + b


def _text_tc(wrows, pid, tt, pos_emb, w01, g, b):
    SUB = 1024
    ROWS = 8
    BM = ROWS * SUB
    NBLK = TOTAL // BM

    def body(w_ref, pid_ref, tt_ref, ptab_ref, w01_ref, g_ref, b_ref, o_ref):
        row0 = w01_ref[0:1, :]
        row1 = w01_ref[1:2, :]
        rhs = jnp.concatenate([ptab_ref[...], row1 - row0, row0], axis=0)
        ones = jnp.ones((1, SUB), jnp.float32)
        for j in range(ROWS):
            pid_row = pid_ref[j:j + 1, :]
            ohT = (pid_row == lax.broadcasted_iota(jnp.int32, (MAXPOS, SUB), 0)
                   ).astype(jnp.float32)
            tt_row = tt_ref[j:j + 1, :].astype(jnp.float32)
            lhs = jnp.concatenate([ohT, tt_row, ones], axis=0)
            add = lax.dot_general(lhs, rhs, (((0,), (0,)), ((), ())),
                                  preferred_element_type=jnp.float32)
            sl = pl.ds(j * SUB, SUB)
            o_ref[sl, :] = _ln_tc(w_ref[sl, :] + add, g_ref[...], b_ref[...])

    row_spec = pl.BlockSpec((1, HID), lambda i: (0, 0))
    return pl.pallas_call(
        body,
        grid=(NBLK,),
        in_specs=[
            pl.BlockSpec((BM, HID), lambda i: (i, 0)),
            pl.BlockSpec((ROWS, SUB), lambda i: (i, 0)),
            pl.BlockSpec((ROWS, SUB), lambda i: (i, 0)),
            pl.BlockSpec((MAXPOS, HID), lambda i: (0, 0)),
            pl.BlockSpec((2, HID), lambda i: (0, 0)),
            row_spec, row_spec,
        ],
        out_specs=pl.BlockSpec((BM, HID), lambda i: (i, 0)),
        out_shape=jax.ShapeDtypeStruct((TOTAL, HID), jnp.float32),
    )(wrows, pid.reshape(TOTAL // SUB, SUB), tt.reshape(TOTAL // SUB, SUB),
      pos_emb, w01, g, b)


def _image_tc(image_flat, img_W, img_b, iln_g, iln_b, w1row, vln_g, vln_b):
    M = B * NB
    BM = 512

    def body(x_ref, w_ref, b_ref, ig_ref, ib_ref, w1_ref, vg_ref, vb_ref, o_ref):
        y = jnp.dot(x_ref[...], w_ref[...], preferred_element_type=jnp.float32)
        y = y + b_ref[...]
        y = _ln_tc(y, ig_ref[...], ib_ref[...])
        y = y + w1_ref[...]
        o_ref[...] = _ln_tc(y, vg_ref[...], vb_ref[...])

    row_spec = pl.BlockSpec((1, HID), lambda i: (0, 0))
    return pl.pallas_call(
        body,
        grid=(M // BM,),
        in_specs=[
            pl.BlockSpec((BM, VDIM), lambda i: (i, 0)),
            pl.BlockSpec((VDIM, HID), lambda i: (0, 0)),
            row_spec, row_spec, row_spec, row_spec, row_spec, row_spec,
        ],
        out_specs=pl.BlockSpec((BM, HID), lambda i: (i, 0)),
        out_shape=jax.ShapeDtypeStruct((M, HID), jnp.float32),
    )(image_flat, img_W, img_b, iln_g, iln_b, w1row, vln_g, vln_b)


def kernel(token_ids, image_feat, token_type_ids, position_ids, word_emb,
           pos_emb, img_W, img_b, ln_g, ln_b, iln_g, iln_b, vln_g, vln_b):
    tid = token_ids.reshape(-1)
    wrows = _word_gather_sc(tid, word_emb)

    r = lambda a: a.reshape(1, HID)
    w01 = lax.slice(word_emb, (0, 0), (2, HID))
    w1row = lax.slice(word_emb, (1, 0), (2, HID))
    img_rows = image_feat.transpose(1, 0, 2).reshape(NB * B, VDIM)
    v = _image_tc(img_rows, img_W, r(img_b),
                  r(iln_g), r(iln_b), w1row, r(vln_g), r(vln_b))
    v = v.reshape(NB, B, HID).transpose(1, 0, 2)

    emb = _text_tc(wrows, position_ids.reshape(-1),
                   token_type_ids.reshape(-1), pos_emb, w01,
                   r(ln_g), r(ln_b)).reshape(B, S, HID)
    return (emb, v)


# speedup vs baseline: 14.5235x; 1.0336x over previous
import functools

import jax
import jax.numpy as jnp
from jax import lax
from jax.experimental import pallas as pl
from jax.experimental.pallas import tpu as pltpu
from jax.experimental.pallas import tpu_sc as plsc

VOCAB = 100000
HID = 128
MAXPOS = 512
VDIM = 2048
B = 1024
S = 200
NB = 36
EPS = 1e-12

NC = 2
NS = 16
NW = NC * NS
TOTAL = B * S
PER_W = TOTAL // NW
CHUNK = 128
NCHUNK = PER_W // CHUNK
NPAIR = NCHUNK // 2


def _word_gather_sc(tid, word_emb):
    mesh = plsc.VectorSubcoreMesh(core_axis_name="c", subcore_axis_name="s")

    @functools.partial(
        pl.kernel,
        out_type=jax.ShapeDtypeStruct((TOTAL, HID), jnp.float32),
        mesh=mesh,
        scratch_types=[
            pltpu.VMEM((2, CHUNK), jnp.int32),
            pltpu.VMEM((2, CHUNK, HID), jnp.float32),
            pltpu.SemaphoreType.DMA,
            pltpu.SemaphoreType.DMA,
            pltpu.SemaphoreType.DMA,
            pltpu.SemaphoreType.DMA,
        ],
    )
    def gather_kernel(tid_h, wtab_h, out_h, tid_v, wbuf, gs0, gs1, os0, os1):
        wid = lax.axis_index("s") * NC + lax.axis_index("c")
        base_w = wid * PER_W
        gsems = (gs0, gs1)
        osems = (os0, os1)

        def gather_desc(b):
            return pltpu.make_async_copy(wtab_h.at[tid_v.at[b]], wbuf.at[b],
                                         gsems[b])

        def fire(c, b):
            base = base_w + c * CHUNK
            pltpu.sync_copy(tid_h.at[pl.ds(base, CHUNK)], tid_v.at[b])
            gather_desc(b).start()

        def out_desc(c, b):
            base = base_w + c * CHUNK
            return pltpu.make_async_copy(wbuf.at[b], out_h.at[pl.ds(base, CHUNK)],
                                         osems[b])

        fire(0, 0)

        def pair_body(p, carry):
            c0 = 2 * p
            c1 = c0 + 1

            @pl.when(p > 0)
            def _():
                out_desc(c0 - 1, 1).wait()

            fire(c1, 1)
            gather_desc(0).wait()
            out_desc(c0, 0).start()

            @pl.when(p + 1 < NPAIR)
            def _():
                out_desc(c0, 0).wait()
                fire(c0 + 2, 0)

            gather_desc(1).wait()
            out_desc(c1, 1).start()
            return carry

        lax.fori_loop(0, NPAIR, pair_body, 0, unroll=False)
        out_desc(NCHUNK - 2, 0).wait()
        out_desc(NCHUNK - 1, 1).wait()

    return gather_kernel(tid, word_emb)


def _ln_tc(y, g, b):
    mu = jnp.mean(y, axis=-1, keepdims=True)
    d = y - mu
    var = jnp.mean(d * d, axis=-1, keepdims=True)
    return d * lax.rsqrt(var + EPS) * g + b


def _text_tc(wrows, pid, tt, pos_emb, w01, g, b):
    SUB = 1024
    ROWS = 8
    BM = ROWS * SUB
    NBLK = TOTAL // BM

    def body(w_ref, pid_ref, tt_ref, ptab_ref, w01_ref, g_ref, b_ref, o_ref):
        row0 = w01_ref[0:1, :]
        row1 = w01_ref[1:2, :]
        dims = (((0,), (0,)), ((), ()))
        for j in range(ROWS):
            pid_row = pid_ref[j:j + 1, :]
            ohT = (pid_row == lax.broadcasted_iota(jnp.int32, (MAXPOS, SUB), 0)
                   ).astype(jnp.float32)
            pos = lax.dot_general(ohT, ptab_ref[...], dims,
                                  preferred_element_type=jnp.float32)
            tt_row = tt_ref[j:j + 1, :].astype(jnp.float32)
            ttemb = lax.dot_general(tt_row, row1 - row0, dims,
                                    preferred_element_type=jnp.float32)
            sl = pl.ds(j * SUB, SUB)
            y = w_ref[sl, :] + pos + (ttemb + row0)
            o_ref[sl, :] = _ln_tc(y, g_ref[...], b_ref[...])

    row_spec = pl.BlockSpec((1, HID), lambda i: (0, 0))
    return pl.pallas_call(
        body,
        grid=(NBLK,),
        in_specs=[
            pl.BlockSpec((BM, HID), lambda i: (i, 0)),
            pl.BlockSpec((ROWS, SUB), lambda i: (i, 0)),
            pl.BlockSpec((ROWS, SUB), lambda i: (i, 0)),
            pl.BlockSpec((MAXPOS, HID), lambda i: (0, 0)),
            pl.BlockSpec((2, HID), lambda i: (0, 0)),
            row_spec, row_spec,
        ],
        out_specs=pl.BlockSpec((BM, HID), lambda i: (i, 0)),
        out_shape=jax.ShapeDtypeStruct((TOTAL, HID), jnp.float32),
    )(wrows, pid.reshape(TOTAL // SUB, SUB), tt.reshape(TOTAL // SUB, SUB),
      pos_emb, w01, g, b)


def _image_tc(image_flat, img_W, img_b, iln_g, iln_b, w1row, vln_g, vln_b):
    M = B * NB
    BM = 1024

    def body(x_ref, w_ref, b_ref, ig_ref, ib_ref, w1_ref, vg_ref, vb_ref, o_ref):
        y = jnp.dot(x_ref[...], w_ref[...], preferred_element_type=jnp.float32)
        y = y + b_ref[...]
        y = _ln_tc(y, ig_ref[...], ib_ref[...])
        y = y + w1_ref[...]
        o_ref[...] = _ln_tc(y, vg_ref[...], vb_ref[...])

    row_spec = pl.BlockSpec((1, HID), lambda i: (0, 0))
    return pl.pallas_call(
        body,
        grid=(M // BM,),
        in_specs=[
            pl.BlockSpec((BM, VDIM), lambda i: (i, 0)),
            pl.BlockSpec((VDIM, HID), lambda i: (0, 0)),
            row_spec, row_spec, row_spec, row_spec, row_spec, row_spec,
        ],
        out_specs=pl.BlockSpec((BM, HID), lambda i: (i, 0)),
        out_shape=jax.ShapeDtypeStruct((M, HID), jnp.float32),
    )(image_flat, img_W, img_b, iln_g, iln_b, w1row, vln_g, vln_b)


def kernel(token_ids, image_feat, token_type_ids, position_ids, word_emb,
           pos_emb, img_W, img_b, ln_g, ln_b, iln_g, iln_b, vln_g, vln_b):
    tid = token_ids.reshape(-1)
    wrows = _word_gather_sc(tid, word_emb)

    r = lambda a: a.reshape(1, HID)
    w01 = lax.slice(word_emb, (0, 0), (2, HID))
    w1row = lax.slice(word_emb, (1, 0), (2, HID))
    img_rows = image_feat.transpose(1, 0, 2).reshape(NB * B, VDIM)
    v = _image_tc(img_rows, img_W, r(img_b),
                  r(iln_g), r(iln_b), w1row, r(vln_g), r(vln_b))
    v = v.reshape(NB, B, HID).transpose(1, 0, 2)

    emb = _text_tc(wrows, position_ids.reshape(-1),
                   token_type_ids.reshape(-1), pos_emb, w01,
                   r(ln_g), r(ln_b)).reshape(B, S, HID)
    return (emb, v)
